# baseline (speedup 1.0000x reference)
import functools

import jax
import jax.numpy as jnp
from jax import lax
from jax.experimental import pallas as pl
from jax.experimental.pallas import tpu as pltpu
from jax.experimental.pallas import tpu_sc as plsc

B, L, V, H = 1024, 200, 1000, 512
N_SPK, SPK_U = 128, 64
EPS = 1e-05

NC, NS = 2, 16
NW = NC * NS
TOKENS = B * L
PER_W = TOKENS // NW
CHUNK = 80
N_CHUNKS = PER_W // CHUNK
NBUF = 3
LAG = 1
FREE = NBUF - LAG
N_PAD = -(-N_CHUNKS // NBUF) * NBUF


def _prep_body(emb_ref, g_ref, b_ref, spk_ids_ref, spk_tab_ref, fc_w_ref,
               fc_b_ref, ln_out_ref, spk_out_ref):
    x = emb_ref[...]
    mean = jnp.mean(x, axis=1, keepdims=True)
    xc = x - mean
    var = jnp.mean(xc * xc, axis=1, keepdims=True)
    ln_out_ref[...] = (xc * lax.rsqrt(var + EPS) * g_ref[0, :][None, :]
                       + b_ref[0, :][None, :])
    sid = spk_ids_ref[...]
    onehot = (sid == lax.broadcasted_iota(jnp.int32, (B, N_SPK), 1))
    se = jnp.dot(onehot.astype(jnp.float32), spk_tab_ref[...],
                 preferred_element_type=jnp.float32)
    feat = jnp.dot(se, fc_w_ref[...], preferred_element_type=jnp.float32)
    feat = feat + fc_b_ref[0, :][None, :]
    spk_out_ref[...] = jax.nn.softplus(feat)


_prep = pl.pallas_call(
    _prep_body,
    out_shape=[
        jax.ShapeDtypeStruct((V, H), jnp.float32),
        jax.ShapeDtypeStruct((B, H), jnp.float32),
    ],
)


_sc_mesh = plsc.VectorSubcoreMesh(core_axis_name="c", subcore_axis_name="s")


@functools.partial(
    pl.kernel,
    out_type=jax.ShapeDtypeStruct((TOKENS, H), jnp.float32),
    mesh=_sc_mesh,
    scratch_types=(
        [pltpu.VMEM((PER_W,), jnp.int32)]
        + [pltpu.VMEM((CHUNK, H), jnp.float32)] * NBUF
        + [pltpu.SemaphoreType.DMA] * (2 * NBUF)
    ),
)
def _sc_gather(table_hbm, idx_hbm, out_hbm, idx_v, *bufs_and_sems):
    bufs = bufs_and_sems[:NBUF]
    gsems = bufs_and_sems[NBUF:2 * NBUF]
    osems = bufs_and_sems[2 * NBUF:]
    sid = lax.axis_index("s")
    wid = sid * NC + lax.axis_index("c")
    base = pl.multiple_of(wid * PER_W, PER_W)
    pltpu.sync_copy(idx_hbm.at[pl.ds(base, PER_W)], idx_v)

    def gat(c, b):
        off = pl.multiple_of(c * CHUNK, CHUNK)
        return pltpu.make_async_copy(
            table_hbm.at[idx_v.at[pl.ds(off, CHUNK)]], bufs[b], gsems[b])

    def put(c, b):
        off = pl.multiple_of(c * CHUNK, CHUNK)
        return pltpu.make_async_copy(bufs[b],
                                     out_hbm.at[pl.ds(base + off, CHUNK)],
                                     osems[b])

    for b in range(LAG):
        gat(b, b).start()

    @pl.loop(0, N_PAD, step=NBUF)
    def _(j):
        for b in range(NBUF):
            c = j + b

            @pl.when(c < N_CHUNKS)
            def _():
                gat(c, b).wait()
                put(c, b).start()
                nb = (b + LAG) % NBUF
                cw = c - FREE

                @pl.when(cw >= 0)
                def _():
                    put(cw, nb).wait()

                cg = c + LAG

                @pl.when(cg < N_CHUNKS)
                def _():
                    gat(cg, nb).start()

    for k in range(N_CHUNKS - FREE, N_CHUNKS):
        put(k, k % NBUF).wait()


def kernel(input_ids, speaker_ids, char_emb, spk_table, fc_w, fc_b, ln_gamma,
           ln_beta):
    ln_table, spk_feat = _prep(char_emb, ln_gamma.reshape(1, H),
                               ln_beta.reshape(1, H), speaker_ids, spk_table,
                               fc_w, fc_b.reshape(1, H))
    flat = _sc_gather(ln_table, input_ids.reshape(TOKENS))
    return flat.reshape(B, L, H), spk_feat.reshape(B, 1, H)


# speedup vs baseline: 1.0130x; 1.0130x over previous
import functools

import jax
import jax.numpy as jnp
from jax import lax
from jax.experimental import pallas as pl
from jax.experimental.pallas import tpu as pltpu
from jax.experimental.pallas import tpu_sc as plsc

B, L, V, H = 1024, 200, 1000, 512
N_SPK, SPK_U = 128, 64
EPS = 1e-05

NC, NS = 2, 16
NW = NC * NS
TOKENS = B * L
PER_W = TOKENS // NW
CHUNK = 40
N_CHUNKS = PER_W // CHUNK
NBUF = 4
LAG = 2
FREE = NBUF - LAG
N_PAD = -(-N_CHUNKS // NBUF) * NBUF


def _prep_body(emb_ref, g_ref, b_ref, spk_ids_ref, spk_tab_ref, fc_w_ref,
               fc_b_ref, ln_out_ref, spk_out_ref):
    x = emb_ref[...]
    mean = jnp.mean(x, axis=1, keepdims=True)
    xc = x - mean
    var = jnp.mean(xc * xc, axis=1, keepdims=True)
    ln_out_ref[...] = (xc * lax.rsqrt(var + EPS) * g_ref[0, :][None, :]
                       + b_ref[0, :][None, :])
    sid = spk_ids_ref[...]
    onehot = (sid == lax.broadcasted_iota(jnp.int32, (B, N_SPK), 1))
    se = jnp.dot(onehot.astype(jnp.float32), spk_tab_ref[...],
                 preferred_element_type=jnp.float32)
    feat = jnp.dot(se, fc_w_ref[...], preferred_element_type=jnp.float32)
    feat = feat + fc_b_ref[0, :][None, :]
    spk_out_ref[...] = jax.nn.softplus(feat)


_prep = pl.pallas_call(
    _prep_body,
    out_shape=[
        jax.ShapeDtypeStruct((V, H), jnp.float32),
        jax.ShapeDtypeStruct((B, H), jnp.float32),
    ],
)


_sc_mesh = plsc.VectorSubcoreMesh(core_axis_name="c", subcore_axis_name="s")


@functools.partial(
    pl.kernel,
    out_type=jax.ShapeDtypeStruct((TOKENS, H), jnp.float32),
    mesh=_sc_mesh,
    scratch_types=(
        [pltpu.VMEM((PER_W,), jnp.int32)]
        + [pltpu.VMEM((CHUNK, H), jnp.float32)] * NBUF
        + [pltpu.SemaphoreType.DMA] * (2 * NBUF)
    ),
)
def _sc_gather(table_hbm, idx_hbm, out_hbm, idx_v, *bufs_and_sems):
    bufs = bufs_and_sems[:NBUF]
    gsems = bufs_and_sems[NBUF:2 * NBUF]
    osems = bufs_and_sems[2 * NBUF:]
    sid = lax.axis_index("s")
    wid = sid * NC + lax.axis_index("c")
    base = pl.multiple_of(wid * PER_W, PER_W)
    pltpu.sync_copy(idx_hbm.at[pl.ds(base, PER_W)], idx_v)

    def gat(c, b):
        off = pl.multiple_of(c * CHUNK, CHUNK)
        return pltpu.make_async_copy(
            table_hbm.at[idx_v.at[pl.ds(off, CHUNK)]], bufs[b], gsems[b])

    def put(c, b):
        off = pl.multiple_of(c * CHUNK, CHUNK)
        return pltpu.make_async_copy(bufs[b],
                                     out_hbm.at[pl.ds(base + off, CHUNK)],
                                     osems[b])

    for b in range(LAG):
        gat(b, b).start()

    @pl.loop(0, N_PAD, step=NBUF)
    def _(j):
        for b in range(NBUF):
            c = j + b

            @pl.when(c < N_CHUNKS)
            def _():
                gat(c, b).wait()
                put(c, b).start()
                nb = (b + LAG) % NBUF
                cw = c - FREE

                @pl.when(cw >= 0)
                def _():
                    put(cw, nb).wait()

                cg = c + LAG

                @pl.when(cg < N_CHUNKS)
                def _():
                    gat(cg, nb).start()

    for k in range(N_CHUNKS - FREE, N_CHUNKS):
        put(k, k % NBUF).wait()


def kernel(input_ids, speaker_ids, char_emb, spk_table, fc_w, fc_b, ln_gamma,
           ln_beta):
    ln_table, spk_feat = _prep(char_emb, ln_gamma.reshape(1, H),
                               ln_beta.reshape(1, H), speaker_ids, spk_table,
                               fc_w, fc_b.reshape(1, H))
    flat = _sc_gather(ln_table, input_ids.reshape(TOKENS))
    return flat.reshape(B, L, H), spk_feat.reshape(B, 1, H)
